# baseline (speedup 1.0000x reference)
import jax
import jax.numpy as jnp
from jax import lax
from jax.experimental import pallas as pl
from jax.experimental.pallas import tpu as pltpu
from jax.experimental.pallas import tpu_sc as plsc

QUEUE = 65536
FEAT = 128
BATCH = 4096
NC = 2
NS = 16
NW = NC * NS
ROWS_W = QUEUE // NW
CHUNK = 256
NCH = ROWS_W // CHUNK
NBUF = 3


def _queue_body(keys, data, ptr, out, ptr_out, *scratch):
    bufs = scratch[0:NBUF]
    lsem = scratch[NBUF:2 * NBUF]
    ssem = scratch[2 * NBUF:3 * NBUF]
    pv = scratch[3 * NBUF]

    wid = lax.axis_index("s") * NC + lax.axis_index("c")

    pltpu.sync_copy(ptr, pv.at[pl.ds(0, 1)])
    praw = pv[...][0]
    pc = jnp.clip(praw, 0, QUEUE - BATCH)

    base = wid * ROWS_W

    for i in range(NCH):
        g = base + i * CHUNK
        in_keys = jnp.logical_and(g >= pc, g < pc + BATCH)

        @pl.when(in_keys)
        def _():
            pltpu.async_copy(
                keys.at[pl.ds(pl.multiple_of(g - pc, 8), CHUNK)],
                out.at[pl.ds(pl.multiple_of(g, 8), CHUNK)], lsem[0])

        @pl.when(jnp.logical_not(in_keys))
        def _():
            pltpu.async_copy(
                data.at[pl.ds(pl.multiple_of(g, 8), CHUNK)],
                out.at[pl.ds(pl.multiple_of(g, 8), CHUNK)], lsem[0])

    for i in range(NCH):
        pltpu.make_async_copy(data.at[pl.ds(0, CHUNK)],
                              out.at[pl.ds(0, CHUNK)], lsem[0]).wait()

    @pl.when(wid == 0)
    def _():
        pv[...] = jnp.full((16,), (praw + BATCH) % QUEUE, jnp.int32)
        pltpu.sync_copy(pv.at[pl.ds(0, 1)], ptr_out)


def kernel(keys, data, ptr):
    mesh = plsc.VectorSubcoreMesh(core_axis_name="c", subcore_axis_name="s")
    f = pl.kernel(
        _queue_body,
        out_type=(
            jax.ShapeDtypeStruct((QUEUE, FEAT), jnp.float32),
            jax.ShapeDtypeStruct((1,), jnp.int32),
        ),
        mesh=mesh,
        scratch_types=(
            [pltpu.VMEM((CHUNK, FEAT), jnp.float32) for _ in range(NBUF)]
            + [pltpu.SemaphoreType.DMA for _ in range(2 * NBUF)]
            + [pltpu.VMEM((16,), jnp.int32)]
        ),
    )
    return f(keys, data, ptr)


# speedup vs baseline: 23.9856x; 23.9856x over previous
import jax
import jax.numpy as jnp
from jax import lax
from jax.experimental import pallas as pl
from jax.experimental.pallas import tpu as pltpu
from jax.experimental.pallas import tpu_sc as plsc

QUEUE = 65536
FEAT = 128
BATCH = 4096
NC = 2
NS = 16
NW = NC * NS
ROWS_W = QUEUE // NW
CHUNK = 256
NCH = ROWS_W // CHUNK
NBUF = 3


def _queue_body(keys, data, ptr, out, ptr_out, *scratch):
    bufs = scratch[0:NBUF]
    lsem = scratch[NBUF:2 * NBUF]
    ssem = scratch[2 * NBUF:3 * NBUF]
    pv = scratch[3 * NBUF]

    wid = lax.axis_index("s") * NC + lax.axis_index("c")

    pltpu.sync_copy(ptr, pv.at[pl.ds(0, 1)])
    praw = pv[...][0]
    pc = jnp.clip(praw, 0, QUEUE - BATCH)

    base = wid * ROWS_W

    def start_load(i):
        b = i % NBUF
        g = base + i * CHUNK
        in_keys = jnp.logical_and(g >= pc, g < pc + BATCH)

        @pl.when(in_keys)
        def _():
            pltpu.async_copy(
                keys.at[pl.ds(pl.multiple_of(g - pc, 8), CHUNK)], bufs[b], lsem[b])

        @pl.when(jnp.logical_not(in_keys))
        def _():
            pltpu.async_copy(
                data.at[pl.ds(pl.multiple_of(g, 8), CHUNK)], bufs[b], lsem[b])

    def wait_load(i):
        b = i % NBUF
        pltpu.make_async_copy(data.at[pl.ds(0, CHUNK)], bufs[b], lsem[b]).wait()

    def start_store(i):
        b = i % NBUF
        g = base + i * CHUNK
        pltpu.async_copy(bufs[b], out.at[pl.ds(pl.multiple_of(g, 8), CHUNK)], ssem[b])

    def wait_store(i):
        b = i % NBUF
        pltpu.make_async_copy(bufs[b], out.at[pl.ds(0, CHUNK)], ssem[b]).wait()

    for i in range(min(NBUF, NCH)):
        start_load(i)
    for i in range(NCH):
        wait_load(i)
        start_store(i)
        ni = i + NBUF
        if ni < NCH:
            wait_store(ni - NBUF)
            start_load(ni)
    for i in range(max(0, NCH - NBUF), NCH):
        wait_store(i)

    @pl.when(wid == 0)
    def _():
        pv[...] = jnp.full((16,), (praw + BATCH) % QUEUE, jnp.int32)
        pltpu.sync_copy(pv.at[pl.ds(0, 1)], ptr_out)


def kernel(keys, data, ptr):
    mesh = plsc.VectorSubcoreMesh(core_axis_name="c", subcore_axis_name="s")
    f = pl.kernel(
        _queue_body,
        out_type=(
            jax.ShapeDtypeStruct((QUEUE, FEAT), jnp.float32),
            jax.ShapeDtypeStruct((1,), jnp.int32),
        ),
        mesh=mesh,
        scratch_types=(
            [pltpu.VMEM((CHUNK, FEAT), jnp.float32) for _ in range(NBUF)]
            + [pltpu.SemaphoreType.DMA for _ in range(2 * NBUF)]
            + [pltpu.VMEM((16,), jnp.int32)]
        ),
    )
    return f(keys, data, ptr)
